# Initial kernel scaffold; baseline (speedup 1.0000x reference)
import jax
import jax.numpy as jnp
from jax import lax
from jax.experimental import pallas as pl
from jax.experimental.pallas import tpu as pltpu
from jax.experimental.pallas import tpu_sc as plsc

_B = 4096
_D = 32
_L = 50
_S = 3 * _L
_NW = 32
_BPW = _B // _NW


def _sc_body(c1_hbm, c2_hbm, c3_hbm, table_hbm, out_hbm,
             c1_v, c2_v, c3_v, r0, r1, o0, o1,
             gsem0, gsem1, osem0, osem1):
    cid = lax.axis_index("c")
    sid = lax.axis_index("s")
    wid = sid * 2 + cid
    base = wid * _BPW

    pltpu.sync_copy(c1_hbm.at[pl.ds(base, _BPW)], c1_v)
    pltpu.sync_copy(c2_hbm.at[pl.ds(base, _BPW)], c2_v)
    pltpu.sync_copy(c3_hbm.at[pl.ds(base, _BPW)], c3_v)

    iota = lax.iota(jnp.int32, 16)

    def fire_gather(i, r, sem):
        pltpu.async_copy(table_hbm.at[c1_v.at[i]], r.at[pl.ds(0, _L)], sem)
        pltpu.async_copy(table_hbm.at[c2_v.at[i]], r.at[pl.ds(_L, _L)], sem)
        pltpu.async_copy(table_hbm.at[c3_v.at[i]], r.at[pl.ds(2 * _L, _L)], sem)

    def wait_gather(i, r, sem):
        pltpu.make_async_copy(table_hbm.at[c1_v.at[i]], r.at[pl.ds(0, _L)], sem).wait()
        pltpu.make_async_copy(table_hbm.at[c2_v.at[i]], r.at[pl.ds(_L, _L)], sem).wait()
        pltpu.make_async_copy(table_hbm.at[c3_v.at[i]], r.at[pl.ds(2 * _L, _L)], sem).wait()

    def transpose(r, o):
        def step(k, carry):
            for u in range(6):
                l = k * 6 + u
                v0 = r[l, pl.ds(0, 16)]
                v1 = r[l, pl.ds(16, 16)]
                lv = jnp.full((16,), 0, jnp.int32) + l
                plsc.store_scatter(o, [iota, lv], v0)
                plsc.store_scatter(o, [iota + 16, lv], v1)
            return carry
        lax.fori_loop(0, _S // 6, step, 0)

    fire_gather(0, r0, gsem0)
    fire_gather(1, r1, gsem1)

    def outer(j, carry):
        i0 = 2 * j
        for b, (r, o, gsem, osem) in enumerate(
                ((r0, o0, gsem0, osem0), (r1, o1, gsem1, osem1))):
            i = i0 + b

            wait_gather(i, r, gsem)

            @pl.when(i0 >= 2)
            def _():
                pltpu.make_async_copy(o, out_hbm.at[base + i - 2], osem).wait()

            transpose(r, o)
            pltpu.async_copy(o, out_hbm.at[base + i], osem)

            @pl.when(j < _BPW // 2 - 1)
            def _():
                fire_gather(i + 2, r, gsem)
        return carry

    lax.fori_loop(0, _BPW // 2, outer, 0)

    pltpu.make_async_copy(o0, out_hbm.at[base + _BPW - 2], osem0).wait()
    pltpu.make_async_copy(o1, out_hbm.at[base + _BPW - 1], osem1).wait()


def kernel(c1_idx, c2_idx, c3_idx, table):
    c1 = c1_idx.astype(jnp.int32)
    c2 = c2_idx.astype(jnp.int32)
    c3 = c3_idx.astype(jnp.int32)
    mesh = plsc.VectorSubcoreMesh(core_axis_name="c", subcore_axis_name="s")
    run = pl.kernel(
        _sc_body,
        mesh=mesh,
        out_type=jax.ShapeDtypeStruct((_B, _D, _S), jnp.float32),
        scratch_types=[
            pltpu.VMEM((_BPW, _L), jnp.int32),
            pltpu.VMEM((_BPW, _L), jnp.int32),
            pltpu.VMEM((_BPW, _L), jnp.int32),
            pltpu.VMEM((_S, _D), jnp.float32),
            pltpu.VMEM((_S, _D), jnp.float32),
            pltpu.VMEM((_D, _S), jnp.float32),
            pltpu.VMEM((_D, _S), jnp.float32),
            pltpu.SemaphoreType.DMA,
            pltpu.SemaphoreType.DMA,
            pltpu.SemaphoreType.DMA,
            pltpu.SemaphoreType.DMA,
        ],
    )
    return run(c1, c2, c3, table)


# speedup vs baseline: 1.4191x; 1.4191x over previous
import jax
import jax.numpy as jnp
from jax import lax
from jax.experimental import pallas as pl
from jax.experimental.pallas import tpu as pltpu
from jax.experimental.pallas import tpu_sc as plsc

_B = 4096
_D = 32
_L = 50
_S = 3 * _L
_NW = 32
_BPW = _B // _NW


def _sc_body(c1_hbm, c2_hbm, c3_hbm, table_hbm, out_hbm,
             c1_v, c2_v, c3_v, r0, r1, o0, o1,
             gsem0, gsem1, osem0, osem1):
    cid = lax.axis_index("c")
    sid = lax.axis_index("s")
    wid = sid * 2 + cid
    base = wid * _BPW

    pltpu.sync_copy(c1_hbm.at[pl.ds(base, _BPW)], c1_v)
    pltpu.sync_copy(c2_hbm.at[pl.ds(base, _BPW)], c2_v)
    pltpu.sync_copy(c3_hbm.at[pl.ds(base, _BPW)], c3_v)

    iota = lax.iota(jnp.int32, 16)

    def fire_gather(i, r, sem):
        pltpu.async_copy(table_hbm.at[c1_v.at[i]], r.at[pl.ds(0, _L)], sem)
        pltpu.async_copy(table_hbm.at[c2_v.at[i]], r.at[pl.ds(_L, _L)], sem)
        pltpu.async_copy(table_hbm.at[c3_v.at[i]], r.at[pl.ds(2 * _L, _L)], sem)

    def wait_gather(i, r, sem):
        pltpu.make_async_copy(table_hbm.at[c1_v.at[i]], r.at[pl.ds(0, _L)], sem).wait()
        pltpu.make_async_copy(table_hbm.at[c2_v.at[i]], r.at[pl.ds(_L, _L)], sem).wait()
        pltpu.make_async_copy(table_hbm.at[c3_v.at[i]], r.at[pl.ds(2 * _L, _L)], sem).wait()

    def transpose(r, o):
        def step(k, carry):
            for u in range(6):
                l = k * 6 + u
                v0 = r[l, pl.ds(0, 16)]
                v1 = r[l, pl.ds(16, 16)]
                lv = jnp.full((16,), 0, jnp.int32) + l
                plsc.store_scatter(o, [iota, lv], v0)
                plsc.store_scatter(o, [iota + 16, lv], v1)
            return carry
        lax.fori_loop(0, _S // 6, step, 0)

    fire_gather(0, r0, gsem0)
    fire_gather(1, r1, gsem1)

    def outer(j, carry):
        i0 = 2 * j
        for b, (r, o, gsem, osem) in enumerate(
                ((r0, o0, gsem0, osem0), (r1, o1, gsem1, osem1))):
            i = i0 + b

            wait_gather(i, r, gsem)

            @pl.when(i0 >= 2)
            def _():
                pltpu.make_async_copy(o, out_hbm.at[base + i - 2], osem).wait()

            transpose(r, o)
            pltpu.async_copy(o, out_hbm.at[base + i], osem)

            @pl.when(j < _BPW // 2 - 1)
            def _():
                fire_gather(i + 2, r, gsem)
        return carry

    lax.fori_loop(0, _BPW // 2, outer, 0)

    pltpu.make_async_copy(o0, out_hbm.at[base + _BPW - 2], osem0).wait()
    pltpu.make_async_copy(o1, out_hbm.at[base + _BPW - 1], osem1).wait()


def kernel(c1_idx, c2_idx, c3_idx, table):
    c1 = c1_idx.astype(jnp.int32)
    c2 = c2_idx.astype(jnp.int32)
    c3 = c3_idx.astype(jnp.int32)
    mesh = plsc.VectorSubcoreMesh(core_axis_name="c", subcore_axis_name="s")
    run = pl.kernel(
        _sc_body,
        mesh=mesh,
        compiler_params=pltpu.CompilerParams(use_tc_tiling_on_sc=False,
                                              needs_layout_passes=False),
        out_type=jax.ShapeDtypeStruct((_B, _D, _S), jnp.float32),
        scratch_types=[
            pltpu.VMEM((_BPW, _L), jnp.int32),
            pltpu.VMEM((_BPW, _L), jnp.int32),
            pltpu.VMEM((_BPW, _L), jnp.int32),
            pltpu.VMEM((_S, _D), jnp.float32),
            pltpu.VMEM((_S, _D), jnp.float32),
            pltpu.VMEM((_D, _S), jnp.float32),
            pltpu.VMEM((_D, _S), jnp.float32),
            pltpu.SemaphoreType.DMA,
            pltpu.SemaphoreType.DMA,
            pltpu.SemaphoreType.DMA,
            pltpu.SemaphoreType.DMA,
        ],
    )
    return run(c1, c2, c3, table)
